# baseline (speedup 1.0000x reference)
import functools
import jax
import jax.numpy as jnp
from jax import lax
from jax.experimental import pallas as pl
from jax.experimental.pallas import tpu as pltpu
from jax.experimental.pallas import tpu_sc as plsc

_NW, _ND, _D = 50000, 10000, 128
_NC, _NS = 2, 16


def _mesh():
    return plsc.VectorSubcoreMesh(core_axis_name="c", subcore_axis_name="s",
                                  num_cores=_NC, num_subcores=_NS)


def _feat_body(x_ref, w_ref, al_ref, ar_ref, fs_ref, el_ref, er_ref, mx_ref):
    j = pl.program_id(1)
    x = x_ref[...]
    w = w_ref[0]
    fs = jnp.dot(x, w, preferred_element_type=jnp.float32)
    fs_ref[0] = fs
    el = jnp.sum(fs * al_ref[0], axis=1)
    er = jnp.sum(fs * ar_ref[0], axis=1)
    el_ref[0, :, 0] = el
    er_ref[0, :, 0] = er

    @pl.when(j == 0)
    def _():
        mx_ref[0] = jnp.full((1, 1), -jnp.inf, jnp.float32)

    mx_ref[0] = jnp.maximum(mx_ref[0], jnp.max(el)[None, None])


def _feat(x, ws, als, ars, blk):
    n = x.shape[0]
    r = ws.shape[0]
    nb = n // blk
    assert n % blk == 0
    return pl.pallas_call(
        _feat_body,
        grid=(r, nb),
        in_specs=[
            pl.BlockSpec((blk, _D), lambda i, j: (j, 0)),
            pl.BlockSpec((1, _D, _D), lambda i, j: (i, 0, 0)),
            pl.BlockSpec((1, 1, _D), lambda i, j: (i, 0, 0)),
            pl.BlockSpec((1, 1, _D), lambda i, j: (i, 0, 0)),
        ],
        out_specs=[
            pl.BlockSpec((1, blk, _D), lambda i, j: (i, j, 0)),
            pl.BlockSpec((1, blk, 1), lambda i, j: (i, j, 0)),
            pl.BlockSpec((1, blk, 1), lambda i, j: (i, j, 0)),
            pl.BlockSpec((1, 1, 1), lambda i, j: (i, 0, 0)),
        ],
        out_shape=[
            jax.ShapeDtypeStruct((r, n, _D), jnp.float32),
            jax.ShapeDtypeStruct((r, n, 1), jnp.float32),
            jax.ShapeDtypeStruct((r, n, 1), jnp.float32),
            jax.ShapeDtypeStruct((r, 1, 1), jnp.float32),
        ],
    )(x, ws, als[:, None, :], ars[:, None, :])


@functools.lru_cache(maxsize=None)
def _make_edge_a(n_src, n_dst, epad, ch, e_total):
    ne = epad // (_NC * _NS)
    nrows = ne // 128
    ndp = _NS * ch

    @functools.partial(
        pl.kernel, mesh=_mesh(), name="gat_edge_a",
        compiler_params=pltpu.CompilerParams(needs_layout_passes=False),
        out_type=[jax.ShapeDtypeStruct((_NC * _NS, nrows, 128), jnp.float32),
                  jax.ShapeDtypeStruct((ndp,), jnp.float32),
                  jax.ShapeDtypeStruct((ndp,), jnp.float32)],
        scratch_types=[pltpu.VMEM((n_src,), jnp.float32),
                       pltpu.VMEM((n_dst,), jnp.float32),
                       pltpu.VMEM((16,), jnp.float32),
                       pltpu.VMEM((ne,), jnp.int32),
                       pltpu.VMEM((nrows, 128), jnp.int32),
                       pltpu.VMEM((nrows, 128), jnp.float32),
                       pltpu.VMEM((ch,), jnp.float32),
                       pltpu.VMEM_SHARED((ndp,), jnp.float32),
                       pltpu.SemaphoreType.DMA],
    )
    def ka(el_h, er_h, mx_h, src_h, dst_h, ee_o, sp0_o, sp1_o,
           elv, erv, mxv, srcv, dstv, eev, zb, s_sh, sem):
        c = lax.axis_index("c")
        t = lax.axis_index("s")
        wid = c * _NS + t
        zeros16 = jnp.zeros((16,), jnp.float32)

        def zb_body(v, carry):
            zb[pl.ds(v * 16, 16)] = zeros16
            return carry

        lax.fori_loop(0, ch // 16, zb_body, 0)
        pltpu.sync_copy(zb, s_sh.at[pl.ds(t * ch, ch)])
        pltpu.sync_copy(el_h, elv)
        pltpu.sync_copy(er_h, erv)
        pltpu.sync_copy(mx_h, mxv)
        pltpu.sync_copy(src_h.at[pl.ds(wid * ne, ne)], srcv)
        pltpu.sync_copy(dst_h.at[wid], dstv)
        plsc.subcore_barrier()
        mx = mxv[...]
        lanes = lax.iota(jnp.int32, 16)

        def row_body(j, carry):
            for k in range(8):
                off = j * 128 + k * 16
                s16 = srcv[pl.ds(off, 16)]
                d16 = dstv[j, pl.ds(k * 16, 16)]
                els = plsc.load_gather(elv, [s16])
                erd = plsc.load_gather(erv, [d16])
                tt = els + erd
                e = jnp.maximum(tt, 0.2 * tt)
                cc = mx + erd
                cd = jnp.maximum(cc, 0.2 * cc)
                ee = jnp.exp(e - cd)
                gidx = wid * ne + off + lanes
                ee = jnp.where(gidx < e_total, ee, 0.0)
                eev[j, pl.ds(k * 16, 16)] = ee
            return carry

        lax.fori_loop(0, nrows, row_body, 0)
        descs = []
        for j in range(nrows):
            descs.append(pltpu.async_copy(
                eev.at[j], s_sh.at[dstv.at[j]], sem, add=True))
        for d in descs:
            d.wait()
        pltpu.sync_copy(eev, ee_o.at[wid])
        plsc.subcore_barrier()

        @pl.when(c == 0)
        def _():
            pltpu.sync_copy(s_sh.at[pl.ds(t * ch, ch)],
                            sp0_o.at[pl.ds(t * ch, ch)])

        @pl.when(c == 1)
        def _():
            pltpu.sync_copy(s_sh.at[pl.ds(t * ch, ch)],
                            sp1_o.at[pl.ds(t * ch, ch)])

    return ka


@functools.lru_cache(maxsize=None)
def _make_edge_b(n_src, n_dst, epad, ch, row_base):
    m = epad // _NS
    mrows = m // 128
    nra = mrows // 2
    ndp = _NS * ch
    rs = ndp // 16
    dr = rs // _NS
    z = dr // 8
    cap = m + 512

    @functools.partial(
        pl.kernel, mesh=_mesh(), name="gat_edge_b",
        compiler_params=pltpu.CompilerParams(needs_layout_passes=False),
        out_type=jax.ShapeDtypeStruct((ndp, _D), jnp.float32),
        scratch_types=[pltpu.VMEM((m,), jnp.int32),
                       pltpu.VMEM((nra, 128), jnp.int32),
                       pltpu.VMEM((nra, 128), jnp.int32),
                       pltpu.VMEM((nra, 128), jnp.float32),
                       pltpu.VMEM((nra, 128), jnp.float32),
                       pltpu.VMEM((cap,), jnp.int32),
                       pltpu.VMEM((cap,), jnp.int32),
                       pltpu.VMEM((cap,), jnp.float32),
                       pltpu.VMEM((64,), jnp.int32),
                       pltpu.VMEM((64,), jnp.int32),
                       pltpu.VMEM((64,), jnp.int32),
                       pltpu.VMEM((64, _D), jnp.float32),
                       pltpu.VMEM((64, _D), jnp.float32),
                       pltpu.VMEM((z, _D), jnp.float32),
                       pltpu.VMEM_SHARED((rs, _D), jnp.float32),
                       pltpu.SemaphoreType.DMA,
                       pltpu.SemaphoreType.DMA],
    )
    def kb(fs_h, src_h, dst_h, ee_h, u_o,
           srcv, dstv0, dstv1, eev0, eev1, csrc, cdst, cee,
           idxg0, idxg1, idxb, rows0, rows1, zb, u_sh, sem0, sem1):
        c = lax.axis_index("c")
        t = lax.axis_index("s")
        e0 = t * m
        zeros16 = jnp.zeros((16,), jnp.float32)

        def zb_body(v, carry):
            for u2 in range(8):
                zb[v, pl.ds(u2 * 16, 16)] = zeros16
            return carry

        lax.fori_loop(0, z, zb_body, 0)
        pltpu.sync_copy(src_h.at[pl.ds(e0, m)], srcv)
        pltpu.sync_copy(dst_h.at[2 * t], dstv0)
        pltpu.sync_copy(dst_h.at[2 * t + 1], dstv1)
        pltpu.sync_copy(ee_h.at[2 * t], eev0)
        pltpu.sync_copy(ee_h.at[2 * t + 1], eev1)
        def range_pass(j, carry0):
            q = c * 8 + j
            lo = q * rs
            zdescs = [pltpu.async_copy(zb, u_sh.at[pl.ds(t * dr + zi * z, z)],
                                       sem0) for zi in range(8)]
            for d in zdescs:
                d.wait()
            plsc.subcore_barrier()
            cnt = 0
            for h, (dstv, eev) in enumerate(((dstv0, eev0), (dstv1, eev1))):

                def scan(rr, cnt):
                    base = (h * nra + rr) * 128
                    for k in range(8):
                        d16 = dstv[rr, pl.ds(k * 16, 16)]
                        e16 = eev[rr, pl.ds(k * 16, 16)]
                        s16 = srcv[pl.ds(base + k * 16, 16)]
                        msk = (d16 >= lo) & (d16 < lo + rs)
                        plsc.store_compressed(cdst.at[pl.ds(cnt, 16)],
                                              d16 - lo, mask=msk)
                        plsc.store_compressed(csrc.at[pl.ds(cnt, 16)],
                                              s16, mask=msk)
                        plsc.store_compressed(cee.at[pl.ds(cnt, 16)],
                                              e16, mask=msk)
                        cnt = cnt + plsc.all_reduce_population_count(msk)[0]
                    return cnt

                cnt = lax.fori_loop(0, nra, scan, cnt)
            zeros16i = jnp.zeros((16,), jnp.int32)
            for k in range(8):
                cdst[pl.ds(cnt + k * 16, 16)] = zeros16i
                cee[pl.ds(cnt + k * 16, 16)] = zeros16
            for k in range(24):
                csrc[pl.ds(cnt + k * 16, 16)] = zeros16i
            npair = (cnt + 127) // 128

            def fire(i, idxg, rows, sem):
                for k in range(4):
                    idxg[pl.ds(k * 16, 16)] = (
                        csrc[pl.ds(i * 64 + k * 16, 16)] + row_base)
                return pltpu.async_copy(fs_h.at[idxg], rows, sem)

            def process(i, idxg, rows, sem):
                pltpu.make_async_copy(fs_h.at[idxg], rows, sem).wait()
                for k in range(4):
                    idxb[pl.ds(k * 16, 16)] = cdst[pl.ds(i * 64 + k * 16, 16)]
                    ev = cee[pl.ds(i * 64 + k * 16, 16)]
                    for qq in range(16):
                        e = k * 16 + qq
                        w = ev[qq]
                        for u2 in range(8):
                            sl = pl.ds(u2 * 16, 16)
                            rows[e, sl] = rows[e, sl] * w
                pltpu.sync_copy(rows, u_sh.at[idxb], add=True)

            fire(0, idxg0, rows0, sem0)
            fire(1, idxg1, rows1, sem1)

            def pair(ii, carry):
                i0 = ii * 2
                process(i0, idxg0, rows0, sem0)
                fire(i0 + 2, idxg0, rows0, sem0)
                process(i0 + 1, idxg1, rows1, sem1)
                fire(i0 + 3, idxg1, rows1, sem1)
                return carry

            lax.fori_loop(0, npair, pair, 0)
            pltpu.make_async_copy(fs_h.at[idxg0], rows0, sem0).wait()
            pltpu.make_async_copy(fs_h.at[idxg1], rows1, sem1).wait()
            plsc.subcore_barrier()
            pltpu.sync_copy(u_sh.at[pl.ds(t * dr, dr)],
                            u_o.at[pl.ds(q * rs + t * dr, dr)])
            return carry0

        lax.fori_loop(0, 8, range_pass, 0)

    return kb


def _comb_body(nrel, *refs):
    urefs = refs[:nrel]
    srefs = refs[nrel:3 * nrel]
    b_ref, o_ref = refs[3 * nrel], refs[3 * nrel + 1]
    j = pl.program_id(0)
    blk = o_ref.shape[0]
    acc = jnp.zeros((blk, _D), jnp.float32)
    del j
    for i in range(nrel):
        acc = acc + b_ref[i][None, :]
        u = urefs[i][...]
        s = srefs[2 * i][...] + srefs[2 * i + 1][...]
        s = jnp.where(s > 0, s, 1.0)
        acc = acc + u / s
    o_ref[...] = jnp.maximum(acc, 0.0)


def _comb(us, ss, bs, n, blk):
    nrel = len(us)
    nb = n // blk
    ndp = us[0].shape[0]
    uspec = pl.BlockSpec((blk, _D), lambda j: (j, 0))
    sspec = pl.BlockSpec((blk, 1), lambda j: (j, 0))
    return pl.pallas_call(
        functools.partial(_comb_body, nrel),
        grid=(nb,),
        in_specs=[uspec] * nrel + [sspec] * (2 * nrel)
        + [pl.BlockSpec((nrel, _D), lambda j: (0, 0))],
        out_specs=pl.BlockSpec((blk, _D), lambda j: (j, 0)),
        out_shape=jax.ShapeDtypeStruct((n, _D), jnp.float32),
    )(*us, *[s[:, None] for s in ss], bs)


def _final_body(h_ref, w_ref, b_ref, o_ref):
    z = jnp.dot(h_ref[...], w_ref[...], preferred_element_type=jnp.float32)
    o_ref[...] = jax.nn.sigmoid(z + b_ref[0, 0])


def _final(h, w_lin, b_lin, blk):
    n = h.shape[0]
    nb = n // blk
    return pl.pallas_call(
        _final_body,
        grid=(nb,),
        in_specs=[pl.BlockSpec((blk, _D), lambda j: (j, 0)),
                  pl.BlockSpec((_D, 1), lambda j: (0, 0)),
                  pl.BlockSpec((1, 1), lambda j: (0, 0))],
        out_specs=pl.BlockSpec((blk, 1), lambda j: (j, 0)),
        out_shape=jax.ShapeDtypeStruct((n, 1), jnp.float32),
    )(h, w_lin, b_lin[None])


_CFG = {
    "ww": (_NW, _NW, 200000, 200704, 3200),
    "wwr": (_NW, _NW, 200000, 200704, 3200),
    "wd": (_NW, _ND, 100000, 102400, 640),
    "dwr": (_ND, _NW, 100000, 102400, 3200),
}


def _pad_edges(src, dst, rel):
    n_src, n_dst, e, epad, _ = _CFG[rel]
    pad = epad - e
    srcp = jnp.concatenate([src.astype(jnp.int32), jnp.zeros((pad,), jnp.int32)])
    dstp = jnp.concatenate([dst.astype(jnp.int32),
                            jnp.full((pad,), n_dst, jnp.int32)])
    return srcp, dstp.reshape(_NC * _NS, epad // (_NC * _NS * 128), 128)


def _edge(rel, el, er, mx, fs_flat, row_base, srcp, dstp2):
    n_src, n_dst, e, epad, ch = _CFG[rel]
    mx16 = jnp.broadcast_to(mx.reshape(1), (16,))
    ka = _make_edge_a(n_src, n_dst, epad, ch, e)
    ee2, sp0, sp1 = ka(el, er, mx16, srcp, dstp2)
    kb = _make_edge_b(n_src, n_dst, epad, ch, row_base)
    u = kb(fs_flat, srcp, dstp2, ee2)
    return u, sp0, sp1


def kernel(x_word, x_doc, ww_src, ww_dst, wwr_src, wwr_dst, wd_src, wd_dst,
           dwr_src, dwr_dst, params):
    edges = {
        "ww": _pad_edges(ww_src, ww_dst, "ww"),
        "wwr": _pad_edges(wwr_src, wwr_dst, "wwr"),
        "wd": _pad_edges(wd_src, wd_dst, "wd"),
        "dwr": _pad_edges(dwr_src, dwr_dst, "dwr"),
    }
    hw, hd = x_word, x_doc
    for l in range(2):
        p = lambda k, rel: params["%s%d_%s" % (k, l, rel)]
        rels_w = ["ww", "wwr", "wd", "dwr"]
        fsg_w, el_w, er_w, mx_w = _feat(
            hw,
            jnp.stack([p("W", r) for r in rels_w]),
            jnp.stack([p("al", r) for r in rels_w]),
            jnp.stack([p("ar", r) for r in rels_w]), 2000)
        rels_d = ["dwr", "wd"]
        fsg_d, el_d, er_d, mx_d = _feat(
            hd,
            jnp.stack([p("W", r) for r in rels_d]),
            jnp.stack([p("al", r) for r in rels_d]),
            jnp.stack([p("ar", r) for r in rels_d]), 2000)
        fw = fsg_w.reshape(4 * _NW, _D)
        fd = fsg_d.reshape(2 * _ND, _D)

        u_ww, *s_ww = _edge("ww", el_w[0, :, 0], er_w[0, :, 0], mx_w[0],
                            fw, 0, *edges["ww"])
        u_wwr, *s_wwr = _edge("wwr", el_w[1, :, 0], er_w[1, :, 0], mx_w[1],
                              fw, _NW, *edges["wwr"])
        u_dwr, *s_dwr = _edge("dwr", el_d[0, :, 0], er_w[3, :, 0], mx_d[0],
                              fd, 0, *edges["dwr"])
        u_wd, *s_wd = _edge("wd", el_w[2, :, 0], er_d[1, :, 0], mx_w[2],
                            fw, 2 * _NW, *edges["wd"])

        hw = _comb([u_ww, u_wwr, u_dwr], [*s_ww, *s_wwr, *s_dwr],
                   jnp.stack([p("b", "ww"), p("b", "wwr"), p("b", "dwr")]),
                   _NW, 2000)
        hd = _comb([u_wd], [*s_wd], jnp.stack([p("b", "wd")]), _ND, 2000)

    ow = _final(hw, params["w_lin"], params["b_lin"], 2000)
    od = _final(hd, params["w_lin"], params["b_lin"], 2000)
    return ow, od


# speedup vs baseline: 1.9883x; 1.9883x over previous
import functools
import jax
import jax.numpy as jnp
from jax import lax
from jax.experimental import pallas as pl
from jax.experimental.pallas import tpu as pltpu
from jax.experimental.pallas import tpu_sc as plsc

_NW, _ND, _D = 50000, 10000, 128
_NC, _NS = 2, 16


def _mesh():
    return plsc.VectorSubcoreMesh(core_axis_name="c", subcore_axis_name="s",
                                  num_cores=_NC, num_subcores=_NS)


def _feat_body(x_ref, w_ref, al_ref, ar_ref, fs_ref, el_ref, er_ref, mx_ref):
    j = pl.program_id(1)
    x = x_ref[...]
    w = w_ref[0]
    fs = jnp.dot(x, w, preferred_element_type=jnp.float32)
    fs_ref[0] = fs
    el = jnp.sum(fs * al_ref[0], axis=1)
    er = jnp.sum(fs * ar_ref[0], axis=1)
    el_ref[0, :, 0] = el
    er_ref[0, :, 0] = er

    @pl.when(j == 0)
    def _():
        mx_ref[0] = jnp.full((1, 1), -jnp.inf, jnp.float32)

    mx_ref[0] = jnp.maximum(mx_ref[0], jnp.max(el)[None, None])


def _feat(x, ws, als, ars, blk):
    n = x.shape[0]
    r = ws.shape[0]
    nb = n // blk
    assert n % blk == 0
    return pl.pallas_call(
        _feat_body,
        grid=(r, nb),
        in_specs=[
            pl.BlockSpec((blk, _D), lambda i, j: (j, 0)),
            pl.BlockSpec((1, _D, _D), lambda i, j: (i, 0, 0)),
            pl.BlockSpec((1, 1, _D), lambda i, j: (i, 0, 0)),
            pl.BlockSpec((1, 1, _D), lambda i, j: (i, 0, 0)),
        ],
        out_specs=[
            pl.BlockSpec((1, blk, _D), lambda i, j: (i, j, 0)),
            pl.BlockSpec((1, blk, 1), lambda i, j: (i, j, 0)),
            pl.BlockSpec((1, blk, 1), lambda i, j: (i, j, 0)),
            pl.BlockSpec((1, 1, 1), lambda i, j: (i, 0, 0)),
        ],
        out_shape=[
            jax.ShapeDtypeStruct((r, n, _D), jnp.float32),
            jax.ShapeDtypeStruct((r, n, 1), jnp.float32),
            jax.ShapeDtypeStruct((r, n, 1), jnp.float32),
            jax.ShapeDtypeStruct((r, 1, 1), jnp.float32),
        ],
    )(x, ws, als[:, None, :], ars[:, None, :])


@functools.lru_cache(maxsize=None)
def _make_edge_a(n_src, n_dst, epad, ch, e_total):
    ne = epad // (_NC * _NS)
    nrows = ne // 128
    ndp = _NS * ch

    @functools.partial(
        pl.kernel, mesh=_mesh(), name="gat_edge_a",
        compiler_params=pltpu.CompilerParams(needs_layout_passes=False),
        out_type=[jax.ShapeDtypeStruct((_NC * _NS, nrows, 128), jnp.float32),
                  jax.ShapeDtypeStruct((ndp,), jnp.float32),
                  jax.ShapeDtypeStruct((ndp,), jnp.float32)],
        scratch_types=[pltpu.VMEM((n_src,), jnp.float32),
                       pltpu.VMEM((n_dst,), jnp.float32),
                       pltpu.VMEM((16,), jnp.float32),
                       pltpu.VMEM((ne,), jnp.int32),
                       pltpu.VMEM((nrows, 128), jnp.int32),
                       pltpu.VMEM((nrows, 128), jnp.float32),
                       pltpu.VMEM((ch,), jnp.float32),
                       pltpu.VMEM_SHARED((ndp,), jnp.float32),
                       pltpu.SemaphoreType.DMA],
    )
    def ka(el_h, er_h, mx_h, src_h, dst_h, ee_o, sp0_o, sp1_o,
           elv, erv, mxv, srcv, dstv, eev, zb, s_sh, sem):
        c = lax.axis_index("c")
        t = lax.axis_index("s")
        wid = c * _NS + t
        zeros16 = jnp.zeros((16,), jnp.float32)

        def zb_body(v, carry):
            zb[pl.ds(v * 16, 16)] = zeros16
            return carry

        lax.fori_loop(0, ch // 16, zb_body, 0)
        pltpu.sync_copy(zb, s_sh.at[pl.ds(t * ch, ch)])
        pltpu.sync_copy(el_h, elv)
        pltpu.sync_copy(er_h, erv)
        pltpu.sync_copy(mx_h, mxv)
        pltpu.sync_copy(src_h.at[pl.ds(wid * ne, ne)], srcv)
        pltpu.sync_copy(dst_h.at[wid], dstv)
        plsc.subcore_barrier()
        mx = mxv[...]
        lanes = lax.iota(jnp.int32, 16)

        def row_body(j, carry):
            for k in range(8):
                off = j * 128 + k * 16
                s16 = srcv[pl.ds(off, 16)]
                d16 = dstv[j, pl.ds(k * 16, 16)]
                els = plsc.load_gather(elv, [s16])
                erd = plsc.load_gather(erv, [d16])
                tt = els + erd
                e = jnp.maximum(tt, 0.2 * tt)
                cc = mx + erd
                cd = jnp.maximum(cc, 0.2 * cc)
                ee = jnp.exp(e - cd)
                gidx = wid * ne + off + lanes
                ee = jnp.where(gidx < e_total, ee, 0.0)
                eev[j, pl.ds(k * 16, 16)] = ee
            return carry

        lax.fori_loop(0, nrows, row_body, 0)
        descs = []
        for j in range(nrows):
            descs.append(pltpu.async_copy(
                eev.at[j], s_sh.at[dstv.at[j]], sem, add=True))
        for d in descs:
            d.wait()
        pltpu.sync_copy(eev, ee_o.at[wid])
        plsc.subcore_barrier()

        @pl.when(c == 0)
        def _():
            pltpu.sync_copy(s_sh.at[pl.ds(t * ch, ch)],
                            sp0_o.at[pl.ds(t * ch, ch)])

        @pl.when(c == 1)
        def _():
            pltpu.sync_copy(s_sh.at[pl.ds(t * ch, ch)],
                            sp1_o.at[pl.ds(t * ch, ch)])

    return ka


@functools.lru_cache(maxsize=None)
def _make_edge_b(n_src, n_dst, epad, ch, row_base):
    m = epad // _NS
    mrows = m // 128
    nra = mrows // 2
    ndp = _NS * ch
    rs = ndp // 16
    dr = rs // _NS
    z = dr // 8
    cap = m + 512

    @functools.partial(
        pl.kernel, mesh=_mesh(), name="gat_edge_b",
        compiler_params=pltpu.CompilerParams(needs_layout_passes=False),
        out_type=jax.ShapeDtypeStruct((ndp, _D), jnp.float32),
        scratch_types=[pltpu.VMEM((m,), jnp.int32),
                       pltpu.VMEM((nra, 128), jnp.int32),
                       pltpu.VMEM((nra, 128), jnp.int32),
                       pltpu.VMEM((nra, 128), jnp.float32),
                       pltpu.VMEM((nra, 128), jnp.float32),
                       pltpu.VMEM((cap,), jnp.int32),
                       pltpu.VMEM((cap,), jnp.int32),
                       pltpu.VMEM((cap,), jnp.float32),
                       pltpu.VMEM((128,), jnp.int32),
                       pltpu.VMEM((128,), jnp.int32),
                       pltpu.VMEM((128,), jnp.int32),
                       pltpu.VMEM((128, _D), jnp.float32),
                       pltpu.VMEM((64, _D), jnp.float32),
                       pltpu.VMEM((z, _D), jnp.float32),
                       pltpu.VMEM_SHARED((rs, _D), jnp.float32),
                       pltpu.SemaphoreType.DMA,
                       pltpu.SemaphoreType.DMA],
    )
    def kb(fs_h, src_h, dst_h, ee_h, u_o,
           srcv, dstv0, dstv1, eev0, eev1, csrc, cdst, cee,
           idxg0, idxg1, idxb, rows0, rows1, zb, u_sh, sem0, sem1):
        c = lax.axis_index("c")
        t = lax.axis_index("s")
        e0 = t * m
        zeros16 = jnp.zeros((16,), jnp.float32)

        def zb_body(v, carry):
            for u2 in range(8):
                zb[v, pl.ds(u2 * 16, 16)] = zeros16
            return carry

        lax.fori_loop(0, z, zb_body, 0)
        pltpu.sync_copy(src_h.at[pl.ds(e0, m)], srcv)
        pltpu.sync_copy(dst_h.at[2 * t], dstv0)
        pltpu.sync_copy(dst_h.at[2 * t + 1], dstv1)
        pltpu.sync_copy(ee_h.at[2 * t], eev0)
        pltpu.sync_copy(ee_h.at[2 * t + 1], eev1)
        def range_pass(j, carry0):
            q = c * 8 + j
            lo = q * rs
            zdescs = [pltpu.async_copy(zb, u_sh.at[pl.ds(t * dr + zi * z, z)],
                                       sem0) for zi in range(8)]
            for d in zdescs:
                d.wait()
            plsc.subcore_barrier()
            cnt = 0
            for h, (dstv, eev) in enumerate(((dstv0, eev0), (dstv1, eev1))):

                def scan(rr, cnt):
                    base = (h * nra + rr) * 128
                    for k in range(8):
                        d16 = dstv[rr, pl.ds(k * 16, 16)]
                        e16 = eev[rr, pl.ds(k * 16, 16)]
                        s16 = srcv[pl.ds(base + k * 16, 16)]
                        msk = (d16 >= lo) & (d16 < lo + rs)
                        plsc.store_compressed(cdst.at[pl.ds(cnt, 16)],
                                              d16 - lo, mask=msk)
                        plsc.store_compressed(csrc.at[pl.ds(cnt, 16)],
                                              s16, mask=msk)
                        plsc.store_compressed(cee.at[pl.ds(cnt, 16)],
                                              e16, mask=msk)
                        cnt = cnt + plsc.all_reduce_population_count(msk)[0]
                    return cnt

                cnt = lax.fori_loop(0, nra, scan, cnt)
            zeros16i = jnp.zeros((16,), jnp.int32)
            for k in range(8):
                cdst[pl.ds(cnt + k * 16, 16)] = zeros16i
                cee[pl.ds(cnt + k * 16, 16)] = zeros16
            for k in range(24):
                csrc[pl.ds(cnt + k * 16, 16)] = zeros16i
            nch = (cnt + 127) // 128

            def chunk(i, carry):
                for k in range(8):
                    idxg0[pl.ds(k * 16, 16)] = (
                        csrc[pl.ds(i * 128 + k * 16, 16)] + row_base)
                pltpu.async_copy(fs_h.at[idxg0], rows0, sem0).wait()
                for k in range(8):
                    idxb[pl.ds(k * 16, 16)] = cdst[pl.ds(i * 128 + k * 16, 16)]
                    ev = cee[pl.ds(i * 128 + k * 16, 16)]
                    for qq in range(16):
                        e = k * 16 + qq
                        w = ev[qq]
                        for u2 in range(8):
                            sl = pl.ds(u2 * 16, 16)
                            rows0[e, sl] = rows0[e, sl] * w
                pltpu.sync_copy(rows0, u_sh.at[idxb], add=True)
                return carry

            lax.fori_loop(0, nch, chunk, 0)
            plsc.subcore_barrier()
            pltpu.sync_copy(u_sh.at[pl.ds(t * dr, dr)],
                            u_o.at[pl.ds(q * rs + t * dr, dr)])
            return carry0

        lax.fori_loop(0, 8, range_pass, 0)

    return kb


def _comb_body(nrel, *refs):
    urefs = refs[:nrel]
    srefs = refs[nrel:3 * nrel]
    b_ref, o_ref = refs[3 * nrel], refs[3 * nrel + 1]
    j = pl.program_id(0)
    blk = o_ref.shape[0]
    acc = jnp.zeros((blk, _D), jnp.float32)
    del j
    for i in range(nrel):
        acc = acc + b_ref[i][None, :]
        u = urefs[i][...]
        s = srefs[2 * i][...] + srefs[2 * i + 1][...]
        s = jnp.where(s > 0, s, 1.0)
        acc = acc + u / s
    o_ref[...] = jnp.maximum(acc, 0.0)


def _comb(us, ss, bs, n, blk):
    nrel = len(us)
    nb = n // blk
    ndp = us[0].shape[0]
    uspec = pl.BlockSpec((blk, _D), lambda j: (j, 0))
    sspec = pl.BlockSpec((blk, 1), lambda j: (j, 0))
    return pl.pallas_call(
        functools.partial(_comb_body, nrel),
        grid=(nb,),
        in_specs=[uspec] * nrel + [sspec] * (2 * nrel)
        + [pl.BlockSpec((nrel, _D), lambda j: (0, 0))],
        out_specs=pl.BlockSpec((blk, _D), lambda j: (j, 0)),
        out_shape=jax.ShapeDtypeStruct((n, _D), jnp.float32),
    )(*us, *[s[:, None] for s in ss], bs)


def _final_body(h_ref, w_ref, b_ref, o_ref):
    z = jnp.dot(h_ref[...], w_ref[...], preferred_element_type=jnp.float32)
    o_ref[...] = jax.nn.sigmoid(z + b_ref[0, 0])


def _final(h, w_lin, b_lin, blk):
    n = h.shape[0]
    nb = n // blk
    return pl.pallas_call(
        _final_body,
        grid=(nb,),
        in_specs=[pl.BlockSpec((blk, _D), lambda j: (j, 0)),
                  pl.BlockSpec((_D, 1), lambda j: (0, 0)),
                  pl.BlockSpec((1, 1), lambda j: (0, 0))],
        out_specs=pl.BlockSpec((blk, 1), lambda j: (j, 0)),
        out_shape=jax.ShapeDtypeStruct((n, 1), jnp.float32),
    )(h, w_lin, b_lin[None])


_CFG = {
    "ww": (_NW, _NW, 200000, 200704, 3200),
    "wwr": (_NW, _NW, 200000, 200704, 3200),
    "wd": (_NW, _ND, 100000, 102400, 640),
    "dwr": (_ND, _NW, 100000, 102400, 3200),
}


def _pad_edges(src, dst, rel):
    n_src, n_dst, e, epad, _ = _CFG[rel]
    pad = epad - e
    srcp = jnp.concatenate([src.astype(jnp.int32), jnp.zeros((pad,), jnp.int32)])
    dstp = jnp.concatenate([dst.astype(jnp.int32),
                            jnp.full((pad,), n_dst, jnp.int32)])
    return srcp, dstp.reshape(_NC * _NS, epad // (_NC * _NS * 128), 128)


def _edge(rel, el, er, mx, fs_flat, row_base, srcp, dstp2):
    n_src, n_dst, e, epad, ch = _CFG[rel]
    mx16 = jnp.broadcast_to(mx.reshape(1), (16,))
    ka = _make_edge_a(n_src, n_dst, epad, ch, e)
    ee2, sp0, sp1 = ka(el, er, mx16, srcp, dstp2)
    kb = _make_edge_b(n_src, n_dst, epad, ch, row_base)
    u = kb(fs_flat, srcp, dstp2, ee2)
    return u, sp0, sp1


def kernel(x_word, x_doc, ww_src, ww_dst, wwr_src, wwr_dst, wd_src, wd_dst,
           dwr_src, dwr_dst, params):
    edges = {
        "ww": _pad_edges(ww_src, ww_dst, "ww"),
        "wwr": _pad_edges(wwr_src, wwr_dst, "wwr"),
        "wd": _pad_edges(wd_src, wd_dst, "wd"),
        "dwr": _pad_edges(dwr_src, dwr_dst, "dwr"),
    }
    hw, hd = x_word, x_doc
    for l in range(2):
        p = lambda k, rel: params["%s%d_%s" % (k, l, rel)]
        rels_w = ["ww", "wwr", "wd", "dwr"]
        fsg_w, el_w, er_w, mx_w = _feat(
            hw,
            jnp.stack([p("W", r) for r in rels_w]),
            jnp.stack([p("al", r) for r in rels_w]),
            jnp.stack([p("ar", r) for r in rels_w]), 2000)
        rels_d = ["dwr", "wd"]
        fsg_d, el_d, er_d, mx_d = _feat(
            hd,
            jnp.stack([p("W", r) for r in rels_d]),
            jnp.stack([p("al", r) for r in rels_d]),
            jnp.stack([p("ar", r) for r in rels_d]), 2000)
        fw = fsg_w.reshape(4 * _NW, _D)
        fd = fsg_d.reshape(2 * _ND, _D)

        u_ww, *s_ww = _edge("ww", el_w[0, :, 0], er_w[0, :, 0], mx_w[0],
                            fw, 0, *edges["ww"])
        u_wwr, *s_wwr = _edge("wwr", el_w[1, :, 0], er_w[1, :, 0], mx_w[1],
                              fw, _NW, *edges["wwr"])
        u_dwr, *s_dwr = _edge("dwr", el_d[0, :, 0], er_w[3, :, 0], mx_d[0],
                              fd, 0, *edges["dwr"])
        u_wd, *s_wd = _edge("wd", el_w[2, :, 0], er_d[1, :, 0], mx_w[2],
                            fw, 2 * _NW, *edges["wd"])

        hw = _comb([u_ww, u_wwr, u_dwr], [*s_ww, *s_wwr, *s_dwr],
                   jnp.stack([p("b", "ww"), p("b", "wwr"), p("b", "dwr")]),
                   _NW, 2000)
        hd = _comb([u_wd], [*s_wd], jnp.stack([p("b", "wd")]), _ND, 2000)

    ow = _final(hw, params["w_lin"], params["b_lin"], 2000)
    od = _final(hd, params["w_lin"], params["b_lin"], 2000)
    return ow, od


# speedup vs baseline: 1.9887x; 1.0002x over previous
import functools
import jax
import jax.numpy as jnp
from jax import lax
from jax.experimental import pallas as pl
from jax.experimental.pallas import tpu as pltpu
from jax.experimental.pallas import tpu_sc as plsc

_NW, _ND, _D = 50000, 10000, 128
_NC, _NS = 2, 16


def _mesh():
    return plsc.VectorSubcoreMesh(core_axis_name="c", subcore_axis_name="s",
                                  num_cores=_NC, num_subcores=_NS)


def _feat_body(x_ref, w_ref, al_ref, ar_ref, fs_ref, el_ref, er_ref, mx_ref):
    j = pl.program_id(1)
    x = x_ref[...]
    w = w_ref[0]
    fs = jnp.dot(x, w, preferred_element_type=jnp.float32)
    fs_ref[0] = fs
    el = jnp.sum(fs * al_ref[0], axis=1)
    er = jnp.sum(fs * ar_ref[0], axis=1)
    el_ref[0, :, 0] = el
    er_ref[0, :, 0] = er

    @pl.when(j == 0)
    def _():
        mx_ref[0] = jnp.full((1, 1), -jnp.inf, jnp.float32)

    mx_ref[0] = jnp.maximum(mx_ref[0], jnp.max(el)[None, None])


def _feat(x, ws, als, ars, blk):
    n = x.shape[0]
    r = ws.shape[0]
    nb = n // blk
    assert n % blk == 0
    return pl.pallas_call(
        _feat_body,
        grid=(r, nb),
        in_specs=[
            pl.BlockSpec((blk, _D), lambda i, j: (j, 0)),
            pl.BlockSpec((1, _D, _D), lambda i, j: (i, 0, 0)),
            pl.BlockSpec((1, 1, _D), lambda i, j: (i, 0, 0)),
            pl.BlockSpec((1, 1, _D), lambda i, j: (i, 0, 0)),
        ],
        out_specs=[
            pl.BlockSpec((1, blk, _D), lambda i, j: (i, j, 0)),
            pl.BlockSpec((1, blk, 1), lambda i, j: (i, j, 0)),
            pl.BlockSpec((1, blk, 1), lambda i, j: (i, j, 0)),
            pl.BlockSpec((1, 1, 1), lambda i, j: (i, 0, 0)),
        ],
        out_shape=[
            jax.ShapeDtypeStruct((r, n, _D), jnp.float32),
            jax.ShapeDtypeStruct((r, n, 1), jnp.float32),
            jax.ShapeDtypeStruct((r, n, 1), jnp.float32),
            jax.ShapeDtypeStruct((r, 1, 1), jnp.float32),
        ],
    )(x, ws, als[:, None, :], ars[:, None, :])


@functools.lru_cache(maxsize=None)
def _make_edge_a(n_src, n_dst, epad, ch, e_total):
    ne = epad // (_NC * _NS)
    nrows = ne // 128
    ndp = _NS * ch

    @functools.partial(
        pl.kernel, mesh=_mesh(), name="gat_edge_a",
        compiler_params=pltpu.CompilerParams(needs_layout_passes=False),
        out_type=[jax.ShapeDtypeStruct((_NC * _NS, nrows, 128), jnp.float32),
                  jax.ShapeDtypeStruct((ndp,), jnp.float32),
                  jax.ShapeDtypeStruct((ndp,), jnp.float32)],
        scratch_types=[pltpu.VMEM((n_src,), jnp.float32),
                       pltpu.VMEM((n_dst,), jnp.float32),
                       pltpu.VMEM((16,), jnp.float32),
                       pltpu.VMEM((ne,), jnp.int32),
                       pltpu.VMEM((nrows, 128), jnp.int32),
                       pltpu.VMEM((nrows, 128), jnp.float32),
                       pltpu.VMEM((ch,), jnp.float32),
                       pltpu.VMEM_SHARED((ndp,), jnp.float32),
                       pltpu.SemaphoreType.DMA],
    )
    def ka(el_h, er_h, mx_h, src_h, dst_h, ee_o, sp0_o, sp1_o,
           elv, erv, mxv, srcv, dstv, eev, zb, s_sh, sem):
        c = lax.axis_index("c")
        t = lax.axis_index("s")
        wid = c * _NS + t
        zeros16 = jnp.zeros((16,), jnp.float32)

        def zb_body(v, carry):
            zb[pl.ds(v * 16, 16)] = zeros16
            return carry

        lax.fori_loop(0, ch // 16, zb_body, 0)
        pltpu.sync_copy(zb, s_sh.at[pl.ds(t * ch, ch)])
        pltpu.sync_copy(el_h, elv)
        pltpu.sync_copy(er_h, erv)
        pltpu.sync_copy(mx_h, mxv)
        pltpu.sync_copy(src_h.at[pl.ds(wid * ne, ne)], srcv)
        pltpu.sync_copy(dst_h.at[wid], dstv)
        plsc.subcore_barrier()
        mx = mxv[...]
        lanes = lax.iota(jnp.int32, 16)

        def row_body(j, carry):
            for k in range(8):
                off = j * 128 + k * 16
                s16 = srcv[pl.ds(off, 16)]
                d16 = dstv[j, pl.ds(k * 16, 16)]
                els = plsc.load_gather(elv, [s16])
                erd = plsc.load_gather(erv, [d16])
                tt = els + erd
                e = jnp.maximum(tt, 0.2 * tt)
                cc = mx + erd
                cd = jnp.maximum(cc, 0.2 * cc)
                ee = jnp.exp(e - cd)
                gidx = wid * ne + off + lanes
                ee = jnp.where(gidx < e_total, ee, 0.0)
                eev[j, pl.ds(k * 16, 16)] = ee
            return carry

        lax.fori_loop(0, nrows, row_body, 0)
        descs = []
        for j in range(nrows):
            descs.append(pltpu.async_copy(
                eev.at[j], s_sh.at[dstv.at[j]], sem, add=True))
        for d in descs:
            d.wait()
        pltpu.sync_copy(eev, ee_o.at[wid])
        plsc.subcore_barrier()

        @pl.when(c == 0)
        def _():
            pltpu.sync_copy(s_sh.at[pl.ds(t * ch, ch)],
                            sp0_o.at[pl.ds(t * ch, ch)])

        @pl.when(c == 1)
        def _():
            pltpu.sync_copy(s_sh.at[pl.ds(t * ch, ch)],
                            sp1_o.at[pl.ds(t * ch, ch)])

    return ka


@functools.lru_cache(maxsize=None)
def _make_edge_b(n_src, n_dst, epad, ch, row_base):
    m = epad // _NS
    mrows = m // 128
    nra = mrows // 2
    ndp = _NS * ch
    rs = ndp // 16
    dr = rs // _NS
    z = dr // 8
    cap = m + 512

    @functools.partial(
        pl.kernel, mesh=_mesh(), name="gat_edge_b",
        compiler_params=pltpu.CompilerParams(needs_layout_passes=False),
        out_type=jax.ShapeDtypeStruct((ndp, _D), jnp.float32),
        scratch_types=[pltpu.VMEM((m,), jnp.int32),
                       pltpu.VMEM((nra, 128), jnp.int32),
                       pltpu.VMEM((nra, 128), jnp.int32),
                       pltpu.VMEM((nra, 128), jnp.float32),
                       pltpu.VMEM((nra, 128), jnp.float32),
                       pltpu.VMEM((cap,), jnp.int32),
                       pltpu.VMEM((cap,), jnp.int32),
                       pltpu.VMEM((cap,), jnp.float32),
                       pltpu.VMEM((128,), jnp.int32),
                       pltpu.VMEM((128,), jnp.int32),
                       pltpu.VMEM((128, _D), jnp.float32),
                       pltpu.VMEM((z, _D), jnp.float32),
                       pltpu.VMEM_SHARED((rs, _D), jnp.float32),
                       pltpu.SemaphoreType.DMA],
    )
    def kb(fs_h, src_h, dst_h, ee_h, u_o,
           srcv, dstv0, dstv1, eev0, eev1, csrc, cdst, cee,
           idxg0, idxb, rows0, zb, u_sh, sem0):
        c = lax.axis_index("c")
        t = lax.axis_index("s")
        e0 = t * m
        zeros16 = jnp.zeros((16,), jnp.float32)

        def zb_body(v, carry):
            for u2 in range(8):
                zb[v, pl.ds(u2 * 16, 16)] = zeros16
            return carry

        lax.fori_loop(0, z, zb_body, 0)
        pltpu.sync_copy(src_h.at[pl.ds(e0, m)], srcv)
        pltpu.sync_copy(dst_h.at[2 * t], dstv0)
        pltpu.sync_copy(dst_h.at[2 * t + 1], dstv1)
        pltpu.sync_copy(ee_h.at[2 * t], eev0)
        pltpu.sync_copy(ee_h.at[2 * t + 1], eev1)
        def range_pass(j, carry0):
            q = c * 8 + j
            lo = q * rs
            zdescs = [pltpu.async_copy(zb, u_sh.at[pl.ds(t * dr + zi * z, z)],
                                       sem0) for zi in range(8)]
            for d in zdescs:
                d.wait()
            plsc.subcore_barrier()
            cnt = 0
            for h, (dstv, eev) in enumerate(((dstv0, eev0), (dstv1, eev1))):

                def scan(rr, cnt):
                    base = (h * nra + rr) * 128
                    for k in range(8):
                        d16 = dstv[rr, pl.ds(k * 16, 16)]
                        e16 = eev[rr, pl.ds(k * 16, 16)]
                        s16 = srcv[pl.ds(base + k * 16, 16)]
                        msk = (d16 >= lo) & (d16 < lo + rs)
                        plsc.store_compressed(cdst.at[pl.ds(cnt, 16)],
                                              d16 - lo, mask=msk)
                        plsc.store_compressed(csrc.at[pl.ds(cnt, 16)],
                                              s16, mask=msk)
                        plsc.store_compressed(cee.at[pl.ds(cnt, 16)],
                                              e16, mask=msk)
                        cnt = cnt + plsc.all_reduce_population_count(msk)[0]
                    return cnt

                cnt = lax.fori_loop(0, nra, scan, cnt)
            zeros16i = jnp.zeros((16,), jnp.int32)
            for k in range(8):
                cdst[pl.ds(cnt + k * 16, 16)] = zeros16i
                cee[pl.ds(cnt + k * 16, 16)] = zeros16
            for k in range(24):
                csrc[pl.ds(cnt + k * 16, 16)] = zeros16i
            nch = (cnt + 127) // 128

            def chunk(i, carry):
                for k in range(8):
                    idxg0[pl.ds(k * 16, 16)] = (
                        csrc[pl.ds(i * 128 + k * 16, 16)] + row_base)
                pltpu.async_copy(fs_h.at[idxg0], rows0, sem0).wait()
                for k in range(8):
                    idxb[pl.ds(k * 16, 16)] = cdst[pl.ds(i * 128 + k * 16, 16)]
                    ev = cee[pl.ds(i * 128 + k * 16, 16)]
                    for qq in range(16):
                        e = k * 16 + qq
                        w = ev[qq]
                        for u2 in range(8):
                            sl = pl.ds(u2 * 16, 16)
                            rows0[e, sl] = rows0[e, sl] * w
                pltpu.sync_copy(rows0, u_sh.at[idxb], add=True)
                return carry

            lax.fori_loop(0, nch, chunk, 0)
            plsc.subcore_barrier()
            pltpu.sync_copy(u_sh.at[pl.ds(t * dr, dr)],
                            u_o.at[pl.ds(q * rs + t * dr, dr)])
            return carry0

        lax.fori_loop(0, 8, range_pass, 0)

    return kb


def _comb_body(nrel, *refs):
    urefs = refs[:nrel]
    srefs = refs[nrel:3 * nrel]
    b_ref, o_ref = refs[3 * nrel], refs[3 * nrel + 1]
    j = pl.program_id(0)
    blk = o_ref.shape[0]
    acc = jnp.zeros((blk, _D), jnp.float32)
    del j
    for i in range(nrel):
        acc = acc + b_ref[i][None, :]
        u = urefs[i][...]
        s = srefs[2 * i][...] + srefs[2 * i + 1][...]
        s = jnp.where(s > 0, s, 1.0)
        acc = acc + u / s
    o_ref[...] = jnp.maximum(acc, 0.0)


def _comb(us, ss, bs, n, blk):
    nrel = len(us)
    nb = n // blk
    ndp = us[0].shape[0]
    uspec = pl.BlockSpec((blk, _D), lambda j: (j, 0))
    sspec = pl.BlockSpec((blk, 1), lambda j: (j, 0))
    return pl.pallas_call(
        functools.partial(_comb_body, nrel),
        grid=(nb,),
        in_specs=[uspec] * nrel + [sspec] * (2 * nrel)
        + [pl.BlockSpec((nrel, _D), lambda j: (0, 0))],
        out_specs=pl.BlockSpec((blk, _D), lambda j: (j, 0)),
        out_shape=jax.ShapeDtypeStruct((n, _D), jnp.float32),
    )(*us, *[s[:, None] for s in ss], bs)


def _final_body(h_ref, w_ref, b_ref, o_ref):
    z = jnp.dot(h_ref[...], w_ref[...], preferred_element_type=jnp.float32)
    o_ref[...] = jax.nn.sigmoid(z + b_ref[0, 0])


def _final(h, w_lin, b_lin, blk):
    n = h.shape[0]
    nb = n // blk
    return pl.pallas_call(
        _final_body,
        grid=(nb,),
        in_specs=[pl.BlockSpec((blk, _D), lambda j: (j, 0)),
                  pl.BlockSpec((_D, 1), lambda j: (0, 0)),
                  pl.BlockSpec((1, 1), lambda j: (0, 0))],
        out_specs=pl.BlockSpec((blk, 1), lambda j: (j, 0)),
        out_shape=jax.ShapeDtypeStruct((n, 1), jnp.float32),
    )(h, w_lin, b_lin[None])


_CFG = {
    "ww": (_NW, _NW, 200000, 200704, 3200),
    "wwr": (_NW, _NW, 200000, 200704, 3200),
    "wd": (_NW, _ND, 100000, 102400, 640),
    "dwr": (_ND, _NW, 100000, 102400, 3200),
}


def _pad_edges(src, dst, rel):
    n_src, n_dst, e, epad, _ = _CFG[rel]
    pad = epad - e
    srcp = jnp.concatenate([src.astype(jnp.int32), jnp.zeros((pad,), jnp.int32)])
    dstp = jnp.concatenate([dst.astype(jnp.int32),
                            jnp.full((pad,), n_dst, jnp.int32)])
    return srcp, dstp.reshape(_NC * _NS, epad // (_NC * _NS * 128), 128)


def _edge(rel, el, er, mx, fs_flat, row_base, srcp, dstp2):
    n_src, n_dst, e, epad, ch = _CFG[rel]
    mx16 = jnp.broadcast_to(mx.reshape(1), (16,))
    ka = _make_edge_a(n_src, n_dst, epad, ch, e)
    ee2, sp0, sp1 = ka(el, er, mx16, srcp, dstp2)
    kb = _make_edge_b(n_src, n_dst, epad, ch, row_base)
    u = kb(fs_flat, srcp, dstp2, ee2)
    return u, sp0, sp1


def kernel(x_word, x_doc, ww_src, ww_dst, wwr_src, wwr_dst, wd_src, wd_dst,
           dwr_src, dwr_dst, params):
    edges = {
        "ww": _pad_edges(ww_src, ww_dst, "ww"),
        "wwr": _pad_edges(wwr_src, wwr_dst, "wwr"),
        "wd": _pad_edges(wd_src, wd_dst, "wd"),
        "dwr": _pad_edges(dwr_src, dwr_dst, "dwr"),
    }
    hw, hd = x_word, x_doc
    for l in range(2):
        p = lambda k, rel: params["%s%d_%s" % (k, l, rel)]
        rels_w = ["ww", "wwr", "wd", "dwr"]
        fsg_w, el_w, er_w, mx_w = _feat(
            hw,
            jnp.stack([p("W", r) for r in rels_w]),
            jnp.stack([p("al", r) for r in rels_w]),
            jnp.stack([p("ar", r) for r in rels_w]), 2000)
        rels_d = ["dwr", "wd"]
        fsg_d, el_d, er_d, mx_d = _feat(
            hd,
            jnp.stack([p("W", r) for r in rels_d]),
            jnp.stack([p("al", r) for r in rels_d]),
            jnp.stack([p("ar", r) for r in rels_d]), 2000)
        fw = fsg_w.reshape(4 * _NW, _D)
        fd = fsg_d.reshape(2 * _ND, _D)

        u_ww, *s_ww = _edge("ww", el_w[0, :, 0], er_w[0, :, 0], mx_w[0],
                            fw, 0, *edges["ww"])
        u_wwr, *s_wwr = _edge("wwr", el_w[1, :, 0], er_w[1, :, 0], mx_w[1],
                              fw, _NW, *edges["wwr"])
        u_dwr, *s_dwr = _edge("dwr", el_d[0, :, 0], er_w[3, :, 0], mx_d[0],
                              fd, 0, *edges["dwr"])
        u_wd, *s_wd = _edge("wd", el_w[2, :, 0], er_d[1, :, 0], mx_w[2],
                            fw, 2 * _NW, *edges["wd"])

        hw = _comb([u_ww, u_wwr, u_dwr], [*s_ww, *s_wwr, *s_dwr],
                   jnp.stack([p("b", "ww"), p("b", "wwr"), p("b", "dwr")]),
                   _NW, 2000)
        hd = _comb([u_wd], [*s_wd], jnp.stack([p("b", "wd")]), _ND, 2000)

    ow = _final(hw, params["w_lin"], params["b_lin"], 2000)
    od = _final(hd, params["w_lin"], params["b_lin"], 2000)
    return ow, od
